# baseline (speedup 1.0000x reference)
import functools

import jax
import jax.numpy as jnp
from jax import lax
from jax.experimental import pallas as pl
from jax.experimental.pallas import tpu as pltpu
from jax.experimental.pallas import tpu_sc as plsc

_N = 10000
_E = 320000
_D = 128
_ALPHA = 0.1
_K = 10

_N_PAD = 10240
_NSUB = 16
_CHUNK = 128
_CH = 176
_CAP = _CH * _CHUNK
_JG = 4
_ROWS_PER_SUB = _N_PAD // _NSUB
_UP_CHUNK = 64
_UP_STEPS = _ROWS_PER_SUB // _UP_CHUNK
_EPR = 8


def _mlp_body(x_ref, w1_ref, b1_ref, w2_ref, b2_ref, o_ref):
    h = jnp.dot(x_ref[...], w1_ref[...], preferred_element_type=jnp.float32)
    h = jnp.maximum(h + b1_ref[...], 0.0)
    o_ref[...] = (
        jnp.dot(h, w2_ref[...], preferred_element_type=jnp.float32) + b2_ref[...]
    )


def _mlp(x_pad, W1, b1, W2, b2):
    blk = 1024
    return pl.pallas_call(
        _mlp_body,
        grid=(_N_PAD // blk,),
        in_specs=[
            pl.BlockSpec((blk, _D), lambda i: (i, 0)),
            pl.BlockSpec((_D, _D), lambda i: (0, 0)),
            pl.BlockSpec((1, _D), lambda i: (0, 0)),
            pl.BlockSpec((_D, _D), lambda i: (0, 0)),
            pl.BlockSpec((1, _D), lambda i: (0, 0)),
        ],
        out_specs=pl.BlockSpec((blk, _D), lambda i: (i, 0)),
        out_shape=jax.ShapeDtypeStruct((_N_PAD, _D), jnp.float32),
    )(x_pad, W1, b1.reshape(1, _D), W2, b2.reshape(1, _D))


def _lane_bcast(v16, u):
    return lax.gather(
        v16,
        jnp.full((16, 1), u, jnp.int32),
        lax.GatherDimensionNumbers(
            offset_dims=(), collapsed_slice_dims=(0,), start_index_map=(0,)),
        slice_sizes=(1,),
        mode=lax.GatherScatterMode.PROMISE_IN_BOUNDS,
    )


def _sc_propagate(hL, hR, ep):
    mesh = plsc.VectorSubcoreMesh(core_axis_name="c", subcore_axis_name="s")
    out_t = [jax.ShapeDtypeStruct((_N_PAD, 64), jnp.float32)] * 2
    scratch = [
        [pltpu.VMEM((3, _CHUNK), jnp.int32) for _ in range(_EPR)],
        [pltpu.VMEM((_CHUNK, 64), jnp.float32) for _ in range(4)],
        pltpu.VMEM((_ROWS_PER_SUB, 64), jnp.float32),
        [pltpu.VMEM((_UP_CHUNK, 64), jnp.float32) for _ in range(2)],
        [pltpu.SemaphoreType.DMA for _ in range(_EPR)],
        [pltpu.SemaphoreType.DMA for _ in range(4)],
        [pltpu.SemaphoreType.DMA for _ in range(2)],
        [pltpu.SemaphoreType.DMA for _ in range(2)],
    ]

    @functools.partial(
        pl.kernel, out_type=out_t, scratch_types=scratch, mesh=mesh,
        compiler_params=pltpu.CompilerParams(
            use_tc_tiling_on_sc=False, needs_layout_passes=False))
    def k(hL_h, hR_h, ep_h, zL_h, zR_h,
          epv, gb, acc, zb, esem, gsem, zrsem, zwsem):
        c = lax.axis_index("c")
        s = lax.axis_index("s")

        def ep_issue(jj, slot):
            pltpu.async_copy(ep_h.at[s, jj], epv[slot], esem[slot])

        def ep_wait_idx(jj, slot):
            pltpu.make_async_copy(ep_h.at[s, jj], epv[slot], esem[slot]).wait()

        def zero_acc_rows(r0, nrows):
            def zrow(i, carry):
                for j in range(_JG):
                    acc[r0 + i, pl.ds(16 * j, 16)] = (
                        jnp.zeros((16,), jnp.float32))
                return carry
            lax.fori_loop(0, nrows, zrow, 0)

        def run(h_hbm, z_hbm):
            base = s * _ROWS_PER_SUB
            zero_acc_rows(0, _ROWS_PER_SUB)

            def init_step(t, carry):
                r0 = base + t * _UP_CHUNK
                pltpu.sync_copy(h_hbm.at[pl.ds(r0, _UP_CHUNK)], zb[0])
                pltpu.sync_copy(zb[0], z_hbm.at[pl.ds(r0, _UP_CHUNK)])
                return carry

            lax.fori_loop(0, _UP_STEPS, init_step, 0)
            plsc.subcore_barrier()

            def gather_issue(slot):
                pltpu.async_copy(
                    z_hbm.at[epv[slot % _EPR].at[0]], gb[slot % 4],
                    gsem[slot % 4])

            def k_step(kk, carry):
                for m in range(5):
                    ep_issue(m, m)
                for m in range(3):
                    ep_wait_idx(m, m)
                    gather_issue(m)

                def group_step(t, carry2):
                    for b in range(_EPR):
                        jj = t * _EPR + b
                        qg = b % 4
                        e_cur = epv[b]
                        gcur = gb[qg]

                        @pl.when(jj + 5 < _CH)
                        def _():
                            ep_issue(jj + 5, (b + 5) % _EPR)

                        @pl.when(jj + 3 < _CH)
                        def _():
                            ep_wait_idx(jj + 3, (b + 3) % _EPR)
                            gather_issue(b + 3)

                        pltpu.make_async_copy(
                            z_hbm.at[e_cur.at[0]], gcur, gsem[qg]).wait()

                        def edge_group(g, carry3):
                            v16 = plsc.bitcast(
                                e_cur[2, pl.ds(g * 16, 16)], jnp.float32)
                            d16 = e_cur[1, pl.ds(g * 16, 16)]
                            for u in range(16):
                                e = g * 16 + u
                                v = _lane_bcast(v16, u)
                                d = d16[u]
                                for j in range(_JG):
                                    sl = pl.ds(16 * j, 16)
                                    acc[d, sl] = (
                                        acc[d, sl] + gcur[e, sl] * v)
                            return carry3

                        lax.fori_loop(0, _CHUNK // 16, edge_group, 0)
                    return carry2

                lax.fori_loop(0, _CH // _EPR, group_step, 0)
                plsc.subcore_barrier()

                def z_read(t, slot):
                    pltpu.async_copy(
                        z_hbm.at[pl.ds(base + t * _UP_CHUNK, _UP_CHUNK)],
                        zb[slot], zrsem[slot])

                def z_read_wait(t, slot):
                    pltpu.make_async_copy(
                        z_hbm.at[pl.ds(base + t * _UP_CHUNK, _UP_CHUNK)],
                        zb[slot], zrsem[slot]).wait()

                z_read(0, 0)

                def up_step(t, carry2):
                    for b in range(2):
                        tt = t * 2 + b
                        r0 = base + tt * _UP_CHUNK
                        a0 = tt * _UP_CHUNK

                        @pl.when(tt + 1 < _UP_STEPS)
                        def _():
                            @pl.when(tt >= 1)
                            def _():
                                pltpu.make_async_copy(
                                    zb[1 - b],
                                    z_hbm.at[pl.ds(r0, _UP_CHUNK)],
                                    zwsem[1 - b]).wait()
                            z_read(tt + 1, 1 - b)

                        z_read_wait(tt, b)

                        def row_step(i, carry3):
                            for j in range(_JG):
                                sl = pl.ds(16 * j, 16)
                                zb[b][i, sl] = (
                                    (1.0 - _ALPHA) * acc[a0 + i, sl]
                                    + _ALPHA * zb[b][i, sl]
                                )
                                acc[a0 + i, sl] = jnp.zeros((16,), jnp.float32)
                            return carry3

                        lax.fori_loop(0, _UP_CHUNK, row_step, 0)
                        pltpu.async_copy(
                            zb[b], z_hbm.at[pl.ds(r0, _UP_CHUNK)], zwsem[b])
                    return carry2

                lax.fori_loop(0, _UP_STEPS // 2, up_step, 0)
                for b in range(2):
                    pltpu.make_async_copy(
                        zb[b], z_hbm.at[pl.ds(base, _UP_CHUNK)], zwsem[b]
                    ).wait()
                plsc.subcore_barrier()
                return carry

            lax.fori_loop(0, _K, k_step, 0)

        @pl.when(c == 0)
        def _():
            run(hL_h, zL_h)

        @pl.when(c == 1)
        def _():
            run(hR_h, zR_h)

    return k(hL, hR, ep)


def kernel(x, edge_index, adj_values, W1, b1, W2, b2):
    x_pad = jnp.zeros((_N_PAD, _D), jnp.float32).at[:_N].set(x)
    h = _mlp(x_pad, W1, b1, W2, b2)
    h = h.at[_N:].set(0.0)

    scale = 1.0 / (1.0 - 0.0 + 1e-05)
    vals = adj_values.astype(jnp.float32) * scale
    dst = edge_index[0].astype(jnp.int32)
    src = edge_index[1].astype(jnp.int32)

    order = jnp.argsort(dst)
    dst_s = dst[order]
    src_s = src[order]
    val_s = vals[order]
    own = dst_s // _ROWS_PER_SUB
    cnt = jnp.bincount(own, length=_NSUB)
    start = jnp.concatenate(
        [jnp.zeros((1,), jnp.int32), jnp.cumsum(cnt)[:-1].astype(jnp.int32)])
    slot = jnp.arange(_E, dtype=jnp.int32) - start[own]
    tgt = own.astype(jnp.int32) * _CAP + slot

    src_slab = jnp.full((_NSUB * _CAP,), _N_PAD - 1, jnp.int32).at[tgt].set(src_s)
    dstl_slab = jnp.zeros((_NSUB * _CAP,), jnp.int32).at[tgt].set(
        dst_s - own.astype(jnp.int32) * _ROWS_PER_SUB)
    val_slab = jnp.zeros((_NSUB * _CAP,), jnp.float32).at[tgt].set(val_s)
    val_i = lax.bitcast_convert_type(val_slab, jnp.int32)
    ep = jnp.stack(
        [src_slab.reshape(_NSUB, _CH, _CHUNK),
         dstl_slab.reshape(_NSUB, _CH, _CHUNK),
         val_i.reshape(_NSUB, _CH, _CHUNK)], axis=2)
    hL = h[:, :64]
    hR = h[:, 64:]
    zL, zR = _sc_propagate(hL, hR, ep)
    out = jnp.concatenate([zL, zR], axis=1)
    return out[:_N]


# speedup vs baseline: 4.2207x; 4.2207x over previous
import functools

import jax
import jax.numpy as jnp
from jax import lax
from jax.experimental import pallas as pl
from jax.experimental.pallas import tpu as pltpu
from jax.experimental.pallas import tpu_sc as plsc

_N = 10000
_E = 320000
_D = 128
_ALPHA = 0.1
_K = 10

_N_PAD = 10240
_NSUB = 16
_CHUNK = 128
_CH = 160
_E_PAD = _NSUB * _CH * _CHUNK
_JG = 4
_ROWS_PER_SUB = _N_PAD // _NSUB
_UP_CHUNK = 64
_UP_STEPS = _ROWS_PER_SUB // _UP_CHUNK
_EPR = 8


def _mlp_body(x_ref, w1_ref, b1_ref, w2_ref, b2_ref, o_ref):
    h = jnp.dot(x_ref[...], w1_ref[...], preferred_element_type=jnp.float32)
    h = jnp.maximum(h + b1_ref[...], 0.0)
    o_ref[...] = (
        jnp.dot(h, w2_ref[...], preferred_element_type=jnp.float32) + b2_ref[...]
    )


def _mlp(x_pad, W1, b1, W2, b2):
    blk = 1024
    return pl.pallas_call(
        _mlp_body,
        grid=(_N_PAD // blk,),
        in_specs=[
            pl.BlockSpec((blk, _D), lambda i: (i, 0)),
            pl.BlockSpec((_D, _D), lambda i: (0, 0)),
            pl.BlockSpec((1, _D), lambda i: (0, 0)),
            pl.BlockSpec((_D, _D), lambda i: (0, 0)),
            pl.BlockSpec((1, _D), lambda i: (0, 0)),
        ],
        out_specs=pl.BlockSpec((blk, _D), lambda i: (i, 0)),
        out_shape=jax.ShapeDtypeStruct((_N_PAD, _D), jnp.float32),
    )(x_pad, W1, b1.reshape(1, _D), W2, b2.reshape(1, _D))


def _lane_bcast(v16, u):
    return lax.gather(
        v16,
        jnp.full((16, 1), u, jnp.int32),
        lax.GatherDimensionNumbers(
            offset_dims=(), collapsed_slice_dims=(0,), start_index_map=(0,)),
        slice_sizes=(1,),
        mode=lax.GatherScatterMode.PROMISE_IN_BOUNDS,
    )


def _sc_propagate(hL, hR, ep):
    mesh = plsc.VectorSubcoreMesh(core_axis_name="c", subcore_axis_name="s")
    out_t = [jax.ShapeDtypeStruct((_N_PAD, 64), jnp.float32)] * 2
    scratch = [
        [pltpu.VMEM((3, _CHUNK), jnp.int32) for _ in range(_EPR)],
        [pltpu.VMEM((_CHUNK, 64), jnp.float32) for _ in range(4)],
        [pltpu.VMEM((_CHUNK, 64), jnp.float32) for _ in range(2)],
        [pltpu.VMEM((_UP_CHUNK, 64), jnp.float32) for _ in range(2)],
        [pltpu.VMEM((_UP_CHUNK, 64), jnp.float32) for _ in range(2)],
        pltpu.VMEM((_UP_CHUNK, 64), jnp.float32),
        pltpu.VMEM_SHARED((_N_PAD, 64), jnp.float32),
        [pltpu.SemaphoreType.DMA for _ in range(_EPR)],
        [pltpu.SemaphoreType.DMA for _ in range(4)],
        [pltpu.SemaphoreType.DMA for _ in range(2)],
        [pltpu.SemaphoreType.DMA for _ in range(2)],
        [pltpu.SemaphoreType.DMA for _ in range(2)],
        [pltpu.SemaphoreType.DMA for _ in range(2)],
    ]

    @functools.partial(
        pl.kernel, out_type=out_t, scratch_types=scratch, mesh=mesh,
        compiler_params=pltpu.CompilerParams(
            use_tc_tiling_on_sc=False, needs_layout_passes=False))
    def k(hL_h, hR_h, ep_h, zL_h, zR_h,
          epv, gb, sb, ab, zb, zzero, agg,
          esem, gsem, ssem, asem, zrsem, zwsem):
        c = lax.axis_index("c")
        s = lax.axis_index("s")
        for i in range(_UP_CHUNK):
            for j in range(_JG):
                zzero[i, pl.ds(16 * j, 16)] = jnp.zeros((16,), jnp.float32)

        def ep_issue(jj, slot):
            pltpu.async_copy(ep_h.at[s, jj], epv[slot], esem[slot])

        def ep_wait(jj, slot):
            pltpu.make_async_copy(ep_h.at[s, jj], epv[slot], esem[slot]).wait()

        def run(h_hbm, z_hbm):
            base = s * _ROWS_PER_SUB

            def init_step(t, carry):
                r0 = base + t * _UP_CHUNK
                pltpu.sync_copy(h_hbm.at[pl.ds(r0, _UP_CHUNK)], zb[0])
                pltpu.sync_copy(zb[0], z_hbm.at[pl.ds(r0, _UP_CHUNK)])
                pltpu.sync_copy(zzero, agg.at[pl.ds(r0, _UP_CHUNK)])
                return carry

            lax.fori_loop(0, _UP_STEPS, init_step, 0)
            plsc.subcore_barrier()

            def gather_issue(jj, slot):
                pltpu.async_copy(
                    z_hbm.at[epv[slot % _EPR].at[0]], gb[slot % 4],
                    gsem[slot % 4])

            def k_step(kk, carry):
                for m in range(5):
                    ep_issue(m, m)
                for m in range(3):
                    ep_wait(m, m)
                    gather_issue(m, m)

                def group_step(t, carry2):
                    for b in range(_EPR):
                        jj = t * _EPR + b
                        qg = b % 4
                        qs = b % 2
                        e_cur = epv[b]
                        gcur, scur = gb[qg], sb[qs]

                        @pl.when(jj + 5 < _CH)
                        def _():
                            ep_issue(jj + 5, (b + 5) % _EPR)

                        @pl.when(jj + 3 < _CH)
                        def _():
                            ep_wait(jj + 3, (b + 3) % _EPR)
                            gather_issue(jj + 3, b + 3)

                        @pl.when(jj >= 2)
                        def _():
                            pltpu.make_async_copy(
                                scur, agg.at[e_cur.at[1]], ssem[qs]).wait()

                        pltpu.make_async_copy(
                            z_hbm.at[e_cur.at[0]], gcur, gsem[qg]).wait()

                        def edge_group(g, carry3):
                            v16 = plsc.bitcast(
                                e_cur[2, pl.ds(g * 16, 16)], jnp.float32)
                            for u in range(16):
                                e = g * 16 + u
                                v = _lane_bcast(v16, u)
                                for j in range(_JG):
                                    sl = pl.ds(16 * j, 16)
                                    scur[e, sl] = gcur[e, sl] * v
                            return carry3

                        lax.fori_loop(0, _CHUNK // 16, edge_group, 0)
                        pltpu.async_copy(
                            scur, agg.at[e_cur.at[1]], ssem[qs], add=True)
                    return carry2

                lax.fori_loop(0, _CH // _EPR, group_step, 0)
                pltpu.make_async_copy(sb[0], agg.at[epv[0].at[1]], ssem[0]).wait()
                pltpu.make_async_copy(sb[1], agg.at[epv[0].at[1]], ssem[1]).wait()
                plsc.subcore_barrier()

                def up_reads(t, slot):
                    r0 = base + t * _UP_CHUNK
                    pltpu.async_copy(
                        agg.at[pl.ds(r0, _UP_CHUNK)], ab[slot], asem[slot])
                    pltpu.async_copy(
                        z_hbm.at[pl.ds(r0, _UP_CHUNK)], zb[slot], zrsem[slot])

                def up_wait_reads(t, slot):
                    r0 = base + t * _UP_CHUNK
                    pltpu.make_async_copy(
                        agg.at[pl.ds(r0, _UP_CHUNK)], ab[slot], asem[slot]
                    ).wait()
                    pltpu.make_async_copy(
                        z_hbm.at[pl.ds(r0, _UP_CHUNK)], zb[slot], zrsem[slot]
                    ).wait()

                up_reads(0, 0)

                def up_step(t, carry2):
                    for b in range(2):
                        tt = t * 2 + b
                        r0 = base + tt * _UP_CHUNK

                        @pl.when(tt + 1 < _UP_STEPS)
                        def _():
                            @pl.when(tt >= 1)
                            def _():
                                pltpu.make_async_copy(
                                    zb[1 - b],
                                    z_hbm.at[pl.ds(r0, _UP_CHUNK)],
                                    zwsem[1 - b]).wait()
                            up_reads(tt + 1, 1 - b)

                        up_wait_reads(tt, b)
                        pltpu.sync_copy(zzero, agg.at[pl.ds(r0, _UP_CHUNK)])

                        def row_step(i, carry3):
                            for j in range(_JG):
                                sl = pl.ds(16 * j, 16)
                                zb[b][i, sl] = (
                                    (1.0 - _ALPHA) * ab[b][i, sl]
                                    + _ALPHA * zb[b][i, sl]
                                )
                            return carry3

                        lax.fori_loop(0, _UP_CHUNK, row_step, 0)
                        pltpu.async_copy(
                            zb[b], z_hbm.at[pl.ds(r0, _UP_CHUNK)], zwsem[b])
                    return carry2

                lax.fori_loop(0, _UP_STEPS // 2, up_step, 0)
                for b in range(2):
                    pltpu.make_async_copy(
                        zb[b], z_hbm.at[pl.ds(base, _UP_CHUNK)], zwsem[b]
                    ).wait()
                plsc.subcore_barrier()
                return carry

            lax.fori_loop(0, _K, k_step, 0)

        @pl.when(c == 0)
        def _():
            run(hL_h, zL_h)

        @pl.when(c == 1)
        def _():
            run(hR_h, zR_h)

    return k(hL, hR, ep)


def kernel(x, edge_index, adj_values, W1, b1, W2, b2):
    x_pad = jnp.zeros((_N_PAD, _D), jnp.float32).at[:_N].set(x)
    h = _mlp(x_pad, W1, b1, W2, b2)
    h = h.at[_N:].set(0.0)

    scale = 1.0 / (1.0 - 0.0 + 1e-05)
    vals = adj_values.astype(jnp.float32) * scale
    dst = edge_index[0].astype(jnp.int32)
    src = edge_index[1].astype(jnp.int32)
    order = jnp.argsort(dst * _N_PAD + src)
    dst = dst[order]
    src = src[order]
    vals = vals[order]
    pad = _E_PAD - _E
    dst_p = jnp.concatenate([dst, jnp.full((pad,), _N_PAD - 1, jnp.int32)])
    src_p = jnp.concatenate([src, jnp.full((pad,), _N_PAD - 1, jnp.int32)])
    val_p = jnp.concatenate([vals, jnp.zeros((pad,), jnp.float32)])
    val_i = lax.bitcast_convert_type(val_p, jnp.int32)
    ep = jnp.stack(
        [src_p.reshape(_NSUB, _CH, _CHUNK),
         dst_p.reshape(_NSUB, _CH, _CHUNK),
         val_i.reshape(_NSUB, _CH, _CHUNK)], axis=2)

    hL = h[:, :64]
    hR = h[:, 64:]
    zL, zR = _sc_propagate(hL, hR, ep)
    out = jnp.concatenate([zL, zR], axis=1)
    return out[:_N]


# speedup vs baseline: 4.8799x; 1.1562x over previous
import functools

import jax
import jax.numpy as jnp
from jax import lax
from jax.experimental import pallas as pl
from jax.experimental.pallas import tpu as pltpu
from jax.experimental.pallas import tpu_sc as plsc

_N = 10000
_E = 320000
_D = 128
_ALPHA = 0.1
_K = 10

_N_PAD = 10240
_NSUB = 16
_CHUNK = 128
_CH = 160
_E_PAD = _NSUB * _CH * _CHUNK
_JG = 4
_ROWS_PER_SUB = _N_PAD // _NSUB
_UP_CHUNK = 64
_UP_STEPS = _ROWS_PER_SUB // _UP_CHUNK
_EPR = 8


def _mlp_body(x_ref, w1_ref, b1_ref, w2_ref, b2_ref, o_ref):
    h = jnp.dot(x_ref[...], w1_ref[...], preferred_element_type=jnp.float32)
    h = jnp.maximum(h + b1_ref[...], 0.0)
    o_ref[...] = (
        jnp.dot(h, w2_ref[...], preferred_element_type=jnp.float32) + b2_ref[...]
    )


def _mlp(x_pad, W1, b1, W2, b2):
    blk = 1024
    return pl.pallas_call(
        _mlp_body,
        grid=(_N_PAD // blk,),
        in_specs=[
            pl.BlockSpec((blk, _D), lambda i: (i, 0)),
            pl.BlockSpec((_D, _D), lambda i: (0, 0)),
            pl.BlockSpec((1, _D), lambda i: (0, 0)),
            pl.BlockSpec((_D, _D), lambda i: (0, 0)),
            pl.BlockSpec((1, _D), lambda i: (0, 0)),
        ],
        out_specs=pl.BlockSpec((blk, _D), lambda i: (i, 0)),
        out_shape=jax.ShapeDtypeStruct((_N_PAD, _D), jnp.float32),
    )(x_pad, W1, b1.reshape(1, _D), W2, b2.reshape(1, _D))


def _lane_bcast(v16, u):
    return lax.gather(
        v16,
        jnp.full((16, 1), u, jnp.int32),
        lax.GatherDimensionNumbers(
            offset_dims=(), collapsed_slice_dims=(0,), start_index_map=(0,)),
        slice_sizes=(1,),
        mode=lax.GatherScatterMode.PROMISE_IN_BOUNDS,
    )


def _sc_propagate(hL, hR, ep):
    mesh = plsc.VectorSubcoreMesh(core_axis_name="c", subcore_axis_name="s")
    out_t = [jax.ShapeDtypeStruct((_N_PAD, 64), jnp.float32)] * 2
    scratch = [
        [pltpu.VMEM((3, _CHUNK), jnp.int32) for _ in range(_EPR)],
        [pltpu.VMEM((_CHUNK, 64), jnp.float32) for _ in range(4)],
        [pltpu.VMEM((_CHUNK, 64), jnp.float32) for _ in range(2)],
        [pltpu.VMEM((_UP_CHUNK, 64), jnp.float32) for _ in range(2)],
        [pltpu.VMEM((_UP_CHUNK, 64), jnp.float32) for _ in range(2)],
        pltpu.VMEM((_UP_CHUNK, 64), jnp.float32),
        pltpu.VMEM_SHARED((_N_PAD, 64), jnp.float32),
        [pltpu.SemaphoreType.DMA for _ in range(_EPR)],
        [pltpu.SemaphoreType.DMA for _ in range(4)],
        [pltpu.SemaphoreType.DMA for _ in range(4)],
        [pltpu.SemaphoreType.DMA for _ in range(2)],
        [pltpu.SemaphoreType.DMA for _ in range(2)],
        [pltpu.SemaphoreType.DMA for _ in range(2)],
        [pltpu.SemaphoreType.DMA for _ in range(2)],
    ]

    @functools.partial(
        pl.kernel, out_type=out_t, scratch_types=scratch, mesh=mesh,
        compiler_params=pltpu.CompilerParams(
            use_tc_tiling_on_sc=False, needs_layout_passes=False))
    def k(hL_h, hR_h, ep_h, zL_h, zR_h,
          epv, gb, sb, ab, zb, zzero, agg,
          esem, gsem, gsem2, ssem, asem, zrsem, zwsem):
        c = lax.axis_index("c")
        s = lax.axis_index("s")
        for i in range(_UP_CHUNK):
            for j in range(_JG):
                zzero[i, pl.ds(16 * j, 16)] = jnp.zeros((16,), jnp.float32)

        def ep_issue(jj, slot):
            pltpu.async_copy(ep_h.at[s, jj], epv[slot], esem[slot])

        def ep_wait(jj, slot):
            pltpu.make_async_copy(ep_h.at[s, jj], epv[slot], esem[slot]).wait()

        def run(h_hbm, z_hbm):
            base = s * _ROWS_PER_SUB

            def init_step(t, carry):
                r0 = base + t * _UP_CHUNK
                pltpu.sync_copy(h_hbm.at[pl.ds(r0, _UP_CHUNK)], zb[0])
                pltpu.sync_copy(zb[0], z_hbm.at[pl.ds(r0, _UP_CHUNK)])
                pltpu.sync_copy(zzero, agg.at[pl.ds(r0, _UP_CHUNK)])
                return carry

            lax.fori_loop(0, _UP_STEPS, init_step, 0)
            plsc.subcore_barrier()

            def gather_issue(jj, slot):
                e_sl = epv[slot % _EPR]
                pltpu.async_copy(
                    z_hbm.at[e_sl.at[0, pl.ds(0, 64)]],
                    gb[slot % 4].at[pl.ds(0, 64)], gsem[slot % 4])
                pltpu.async_copy(
                    z_hbm.at[e_sl.at[0, pl.ds(64, 64)]],
                    gb[slot % 4].at[pl.ds(64, 64)], gsem2[slot % 4])

            def k_step(kk, carry):
                for m in range(5):
                    ep_issue(m, m)
                for m in range(3):
                    ep_wait(m, m)
                    gather_issue(m, m)

                def group_step(t, carry2):
                    for b in range(_EPR):
                        jj = t * _EPR + b
                        qg = b % 4
                        qs = b % 2
                        e_cur = epv[b]
                        gcur, scur = gb[qg], sb[qs]

                        @pl.when(jj + 5 < _CH)
                        def _():
                            ep_issue(jj + 5, (b + 5) % _EPR)

                        @pl.when(jj + 3 < _CH)
                        def _():
                            ep_wait(jj + 3, (b + 3) % _EPR)
                            gather_issue(jj + 3, b + 3)

                        @pl.when(jj >= 2)
                        def _():
                            pltpu.make_async_copy(
                                scur, agg.at[e_cur.at[1]], ssem[qs]).wait()

                        pltpu.make_async_copy(
                            z_hbm.at[e_cur.at[0, pl.ds(0, 64)]],
                            gcur.at[pl.ds(0, 64)], gsem[qg]).wait()
                        pltpu.make_async_copy(
                            z_hbm.at[e_cur.at[0, pl.ds(64, 64)]],
                            gcur.at[pl.ds(64, 64)], gsem2[qg]).wait()

                        def edge_group(g, carry3):
                            v16 = plsc.bitcast(
                                e_cur[2, pl.ds(g * 16, 16)], jnp.float32)
                            for u in range(16):
                                e = g * 16 + u
                                v = _lane_bcast(v16, u)
                                for j in range(_JG):
                                    sl = pl.ds(16 * j, 16)
                                    scur[e, sl] = gcur[e, sl] * v
                            return carry3

                        lax.fori_loop(0, _CHUNK // 16, edge_group, 0)
                        pltpu.async_copy(
                            scur, agg.at[e_cur.at[1]], ssem[qs], add=True)
                    return carry2

                lax.fori_loop(0, _CH // _EPR, group_step, 0)
                pltpu.make_async_copy(sb[0], agg.at[epv[0].at[1]], ssem[0]).wait()
                pltpu.make_async_copy(sb[1], agg.at[epv[0].at[1]], ssem[1]).wait()
                plsc.subcore_barrier()

                def up_reads(t, slot):
                    r0 = base + t * _UP_CHUNK
                    pltpu.async_copy(
                        agg.at[pl.ds(r0, _UP_CHUNK)], ab[slot], asem[slot])
                    pltpu.async_copy(
                        z_hbm.at[pl.ds(r0, _UP_CHUNK)], zb[slot], zrsem[slot])

                def up_wait_reads(t, slot):
                    r0 = base + t * _UP_CHUNK
                    pltpu.make_async_copy(
                        agg.at[pl.ds(r0, _UP_CHUNK)], ab[slot], asem[slot]
                    ).wait()
                    pltpu.make_async_copy(
                        z_hbm.at[pl.ds(r0, _UP_CHUNK)], zb[slot], zrsem[slot]
                    ).wait()

                up_reads(0, 0)

                def up_step(t, carry2):
                    for b in range(2):
                        tt = t * 2 + b
                        r0 = base + tt * _UP_CHUNK

                        @pl.when(tt + 1 < _UP_STEPS)
                        def _():
                            @pl.when(tt >= 1)
                            def _():
                                pltpu.make_async_copy(
                                    zb[1 - b],
                                    z_hbm.at[pl.ds(r0, _UP_CHUNK)],
                                    zwsem[1 - b]).wait()
                            up_reads(tt + 1, 1 - b)

                        up_wait_reads(tt, b)
                        pltpu.sync_copy(zzero, agg.at[pl.ds(r0, _UP_CHUNK)])

                        def row_step(i, carry3):
                            for j in range(_JG):
                                sl = pl.ds(16 * j, 16)
                                zb[b][i, sl] = (
                                    (1.0 - _ALPHA) * ab[b][i, sl]
                                    + _ALPHA * zb[b][i, sl]
                                )
                            return carry3

                        lax.fori_loop(0, _UP_CHUNK, row_step, 0)
                        pltpu.async_copy(
                            zb[b], z_hbm.at[pl.ds(r0, _UP_CHUNK)], zwsem[b])
                    return carry2

                lax.fori_loop(0, _UP_STEPS // 2, up_step, 0)
                for b in range(2):
                    pltpu.make_async_copy(
                        zb[b], z_hbm.at[pl.ds(base, _UP_CHUNK)], zwsem[b]
                    ).wait()
                plsc.subcore_barrier()
                return carry

            lax.fori_loop(0, _K, k_step, 0)

        @pl.when(c == 0)
        def _():
            run(hL_h, zL_h)

        @pl.when(c == 1)
        def _():
            run(hR_h, zR_h)

    return k(hL, hR, ep)


def kernel(x, edge_index, adj_values, W1, b1, W2, b2):
    x_pad = jnp.zeros((_N_PAD, _D), jnp.float32).at[:_N].set(x)
    h = _mlp(x_pad, W1, b1, W2, b2)
    h = h.at[_N:].set(0.0)

    scale = 1.0 / (1.0 - 0.0 + 1e-05)
    vals = adj_values.astype(jnp.float32) * scale
    dst = edge_index[0].astype(jnp.int32)
    src = edge_index[1].astype(jnp.int32)
    order = jnp.argsort(dst)
    dst = dst[order]
    src = src[order]
    vals = vals[order]
    pad = _E_PAD - _E
    dst_p = jnp.concatenate([dst, jnp.full((pad,), _N_PAD - 1, jnp.int32)])
    src_p = jnp.concatenate([src, jnp.full((pad,), _N_PAD - 1, jnp.int32)])
    val_p = jnp.concatenate([vals, jnp.zeros((pad,), jnp.float32)])
    val_i = lax.bitcast_convert_type(val_p, jnp.int32)
    ep = jnp.stack(
        [src_p.reshape(_NSUB, _CH, _CHUNK),
         dst_p.reshape(_NSUB, _CH, _CHUNK),
         val_i.reshape(_NSUB, _CH, _CHUNK)], axis=2)

    hL = h[:, :64]
    hR = h[:, 64:]
    zL, zR = _sc_propagate(hL, hR, ep)
    out = jnp.concatenate([zL, zR], axis=1)
    return out[:_N]


# speedup vs baseline: 4.8839x; 1.0008x over previous
import functools

import jax
import jax.numpy as jnp
from jax import lax
from jax.experimental import pallas as pl
from jax.experimental.pallas import tpu as pltpu
from jax.experimental.pallas import tpu_sc as plsc

_N = 10000
_E = 320000
_D = 128
_ALPHA = 0.1
_K = 10

_N_PAD = 10240
_NSUB = 16
_CHUNK = 128
_CH = 160
_E_PAD = _NSUB * _CH * _CHUNK
_JG = 4
_ROWS_PER_SUB = _N_PAD // _NSUB
_UP_CHUNK = 64
_UP_STEPS = _ROWS_PER_SUB // _UP_CHUNK
_EPR = 8


def _mlp_body(x_ref, w1_ref, b1_ref, w2_ref, b2_ref, o_ref):
    h = jnp.dot(x_ref[...], w1_ref[...], preferred_element_type=jnp.float32)
    h = jnp.maximum(h + b1_ref[...], 0.0)
    o_ref[...] = (
        jnp.dot(h, w2_ref[...], preferred_element_type=jnp.float32) + b2_ref[...]
    )


def _mlp(x_pad, W1, b1, W2, b2):
    blk = 1024
    return pl.pallas_call(
        _mlp_body,
        grid=(_N_PAD // blk,),
        in_specs=[
            pl.BlockSpec((blk, _D), lambda i: (i, 0)),
            pl.BlockSpec((_D, _D), lambda i: (0, 0)),
            pl.BlockSpec((1, _D), lambda i: (0, 0)),
            pl.BlockSpec((_D, _D), lambda i: (0, 0)),
            pl.BlockSpec((1, _D), lambda i: (0, 0)),
        ],
        out_specs=pl.BlockSpec((blk, _D), lambda i: (i, 0)),
        out_shape=jax.ShapeDtypeStruct((_N_PAD, _D), jnp.float32),
    )(x_pad, W1, b1.reshape(1, _D), W2, b2.reshape(1, _D))


def _lane_bcast(v16, u):
    return lax.gather(
        v16,
        jnp.full((16, 1), u, jnp.int32),
        lax.GatherDimensionNumbers(
            offset_dims=(), collapsed_slice_dims=(0,), start_index_map=(0,)),
        slice_sizes=(1,),
        mode=lax.GatherScatterMode.PROMISE_IN_BOUNDS,
    )


def _sc_propagate(hL, hR, ep):
    mesh = plsc.VectorSubcoreMesh(core_axis_name="c", subcore_axis_name="s")
    out_t = [jax.ShapeDtypeStruct((_N_PAD, 64), jnp.float32)] * 2
    scratch = [
        [pltpu.VMEM((3, _CHUNK), jnp.int32) for _ in range(_EPR)],
        [pltpu.VMEM((_CHUNK, 64), jnp.float32) for _ in range(4)],
        [pltpu.VMEM((_CHUNK, 64), jnp.float32) for _ in range(2)],
        [pltpu.VMEM((_UP_CHUNK, 64), jnp.float32) for _ in range(2)],
        [pltpu.VMEM((_UP_CHUNK, 64), jnp.float32) for _ in range(2)],
        pltpu.VMEM((_UP_CHUNK, 64), jnp.float32),
        pltpu.VMEM_SHARED((_N_PAD, 64), jnp.float32),
        [pltpu.SemaphoreType.DMA for _ in range(_EPR)],
        [pltpu.SemaphoreType.DMA for _ in range(4)],
        [pltpu.SemaphoreType.DMA for _ in range(2)],
        [pltpu.SemaphoreType.DMA for _ in range(2)],
        [pltpu.SemaphoreType.DMA for _ in range(2)],
        [pltpu.SemaphoreType.DMA for _ in range(2)],
    ]

    @functools.partial(
        pl.kernel, out_type=out_t, scratch_types=scratch, mesh=mesh,
        compiler_params=pltpu.CompilerParams(
            use_tc_tiling_on_sc=False, needs_layout_passes=False))
    def k(hL_h, hR_h, ep_h, zL_h, zR_h,
          epv, gb, sb, ab, zb, zzero, agg,
          esem, gsem, ssem, asem, zrsem, zwsem):
        c = lax.axis_index("c")
        s = lax.axis_index("s")
        for i in range(_UP_CHUNK):
            for j in range(_JG):
                zzero[i, pl.ds(16 * j, 16)] = jnp.zeros((16,), jnp.float32)

        def ep_issue(jj, slot):
            pltpu.async_copy(ep_h.at[s, jj], epv[slot], esem[slot])

        def ep_wait(jj, slot):
            pltpu.make_async_copy(ep_h.at[s, jj], epv[slot], esem[slot]).wait()

        def run(h_hbm, z_hbm):
            base = s * _ROWS_PER_SUB

            def init_step(t, carry):
                r0 = base + t * _UP_CHUNK
                pltpu.sync_copy(h_hbm.at[pl.ds(r0, _UP_CHUNK)], zb[0])
                pltpu.sync_copy(zb[0], z_hbm.at[pl.ds(r0, _UP_CHUNK)])
                pltpu.sync_copy(zzero, agg.at[pl.ds(r0, _UP_CHUNK)])
                return carry

            lax.fori_loop(0, _UP_STEPS, init_step, 0)
            plsc.subcore_barrier()

            def gather_issue(jj, slot):
                pltpu.async_copy(
                    z_hbm.at[epv[slot % _EPR].at[0]], gb[slot % 4],
                    gsem[slot % 4])

            def k_step(kk, carry):
                for m in range(5):
                    ep_issue(m, m)
                for m in range(3):
                    ep_wait(m, m)
                    gather_issue(m, m)

                def group_step(t, carry2):
                    for b in range(_EPR):
                        jj = t * _EPR + b
                        qg = b % 4
                        qs = b % 2
                        e_cur = epv[b]
                        gcur, scur = gb[qg], sb[qs]

                        @pl.when(jj + 5 < _CH)
                        def _():
                            ep_issue(jj + 5, (b + 5) % _EPR)

                        @pl.when(jj + 3 < _CH)
                        def _():
                            ep_wait(jj + 3, (b + 3) % _EPR)
                            gather_issue(jj + 3, b + 3)

                        @pl.when(jj >= 2)
                        def _():
                            pltpu.make_async_copy(
                                scur, agg.at[e_cur.at[1]], ssem[qs]).wait()

                        pltpu.make_async_copy(
                            z_hbm.at[e_cur.at[0]], gcur, gsem[qg]).wait()

                        def edge_group(g, carry3):
                            v16 = plsc.bitcast(
                                e_cur[2, pl.ds(g * 16, 16)], jnp.float32)
                            for u in range(16):
                                e = g * 16 + u
                                v = _lane_bcast(v16, u)
                                for j in range(_JG):
                                    sl = pl.ds(16 * j, 16)
                                    scur[e, sl] = gcur[e, sl] * v
                            return carry3

                        lax.fori_loop(0, _CHUNK // 16, edge_group, 0)
                        pltpu.async_copy(
                            scur, agg.at[e_cur.at[1]], ssem[qs], add=True)
                    return carry2

                lax.fori_loop(0, _CH // _EPR, group_step, 0)
                pltpu.make_async_copy(sb[0], agg.at[epv[0].at[1]], ssem[0]).wait()
                pltpu.make_async_copy(sb[1], agg.at[epv[0].at[1]], ssem[1]).wait()
                plsc.subcore_barrier()

                def up_reads(t, slot):
                    r0 = base + t * _UP_CHUNK
                    pltpu.async_copy(
                        agg.at[pl.ds(r0, _UP_CHUNK)], ab[slot], asem[slot])
                    pltpu.async_copy(
                        z_hbm.at[pl.ds(r0, _UP_CHUNK)], zb[slot], zrsem[slot])

                def up_wait_reads(t, slot):
                    r0 = base + t * _UP_CHUNK
                    pltpu.make_async_copy(
                        agg.at[pl.ds(r0, _UP_CHUNK)], ab[slot], asem[slot]
                    ).wait()
                    pltpu.make_async_copy(
                        z_hbm.at[pl.ds(r0, _UP_CHUNK)], zb[slot], zrsem[slot]
                    ).wait()

                up_reads(0, 0)

                def up_step(t, carry2):
                    for b in range(2):
                        tt = t * 2 + b
                        r0 = base + tt * _UP_CHUNK

                        @pl.when(tt + 1 < _UP_STEPS)
                        def _():
                            @pl.when(tt >= 1)
                            def _():
                                pltpu.make_async_copy(
                                    zb[1 - b],
                                    z_hbm.at[pl.ds(r0, _UP_CHUNK)],
                                    zwsem[1 - b]).wait()
                            up_reads(tt + 1, 1 - b)

                        up_wait_reads(tt, b)
                        pltpu.sync_copy(zzero, agg.at[pl.ds(r0, _UP_CHUNK)])

                        def row_step(i, carry3):
                            for j in range(_JG):
                                sl = pl.ds(16 * j, 16)
                                zb[b][i, sl] = (
                                    (1.0 - _ALPHA) * ab[b][i, sl]
                                    + _ALPHA * zb[b][i, sl]
                                )
                            return carry3

                        lax.fori_loop(0, _UP_CHUNK, row_step, 0)
                        pltpu.async_copy(
                            zb[b], z_hbm.at[pl.ds(r0, _UP_CHUNK)], zwsem[b])
                    return carry2

                lax.fori_loop(0, _UP_STEPS // 2, up_step, 0)
                for b in range(2):
                    pltpu.make_async_copy(
                        zb[b], z_hbm.at[pl.ds(base, _UP_CHUNK)], zwsem[b]
                    ).wait()
                plsc.subcore_barrier()
                return carry

            lax.fori_loop(0, _K, k_step, 0)

        @pl.when(c == 0)
        def _():
            run(hL_h, zL_h)

        @pl.when(c == 1)
        def _():
            run(hR_h, zR_h)

    return k(hL, hR, ep)


def kernel(x, edge_index, adj_values, W1, b1, W2, b2):
    x_pad = jnp.zeros((_N_PAD, _D), jnp.float32).at[:_N].set(x)
    h = _mlp(x_pad, W1, b1, W2, b2)
    h = h.at[_N:].set(0.0)

    scale = 1.0 / (1.0 - 0.0 + 1e-05)
    vals = adj_values.astype(jnp.float32) * scale
    dst = edge_index[0].astype(jnp.int32)
    src = edge_index[1].astype(jnp.int32)
    order = jnp.argsort(dst)
    dst = dst[order]
    src = src[order]
    vals = vals[order]
    pad = _E_PAD - _E
    dst_p = jnp.concatenate([dst, jnp.full((pad,), _N_PAD - 1, jnp.int32)])
    src_p = jnp.concatenate([src, jnp.full((pad,), _N_PAD - 1, jnp.int32)])
    val_p = jnp.concatenate([vals, jnp.zeros((pad,), jnp.float32)])
    val_i = lax.bitcast_convert_type(val_p, jnp.int32)
    ep = jnp.stack(
        [src_p.reshape(_NSUB, _CH, _CHUNK),
         dst_p.reshape(_NSUB, _CH, _CHUNK),
         val_i.reshape(_NSUB, _CH, _CHUNK)], axis=2)

    hL = h[:, :64]
    hR = h[:, 64:]
    zL, zR = _sc_propagate(hL, hR, ep)
    out = jnp.concatenate([zL, zR], axis=1)
    return out[:_N]
